# baseline (speedup 1.0000x reference)
import jax
import jax.numpy as jnp
from jax import lax
from jax.experimental import pallas as pl
from jax.experimental.pallas import tpu as pltpu
from jax.experimental.pallas import tpu_sc as plsc

N_NODES = 10000
N_PAD = 10240
N_EDGES = 320000
D_IN = 128
D_HALF = 64
D_OUT = 256
N_CLASSES = 2
NC = 2
NS = 16
ROWS_T = N_PAD // NS
E_T = N_EDGES // NS
CHUNK = 80
NCHUNK = E_T // CHUNK
NQUAD = NCHUNK // 4
DEG_F = 10
XROWS = 160
PART = N_NODES - (NS - 1) * ROWS_T - 2 * XROWS


def _sc_body(xs, er, out, h_hbm,
             sid_v, did_v, rows_v, xbuf, ns_v, nd_v, db_v, ones_v,
             agg_sh, dego_sh, degi_sh, gsem, ssem, dsem):
    c = lax.axis_index("c")
    t = lax.axis_index("s")
    r0 = t * ROWS_T

    _sc0 = jax.named_scope("p0_setup"); _sc0.__enter__()
    pltpu.sync_copy(er.at[0, t], sid_v)
    pltpu.sync_copy(er.at[1, t], did_v)

    def zero_xbuf(i, carry):
        for j in range(D_HALF // 16):
            xbuf[i, pl.ds(j * 16, 16)] = jnp.zeros((16,), jnp.float32)
        return carry
    lax.fori_loop(0, XROWS, zero_xbuf, 0)

    def zero_db(i, carry):
        db_v[pl.ds(i * 16, 16)] = jnp.zeros((16,), jnp.float32)
        return carry
    lax.fori_loop(0, ROWS_T // 16, zero_db, 0)

    def fill_ones(i, carry):
        ones_v[pl.ds(i * 16, 16)] = jnp.full((16,), 1.0, jnp.float32)
        return carry
    lax.fori_loop(0, CHUNK // 16, fill_ones, 0)

    for q in range(ROWS_T // XROWS):
        pltpu.sync_copy(xbuf, agg_sh.at[pl.ds(r0 + q * XROWS, XROWS)])
    pltpu.sync_copy(db_v, dego_sh.at[pl.ds(r0, ROWS_T)])
    pltpu.sync_copy(db_v, degi_sh.at[pl.ds(r0, ROWS_T)])
    plsc.subcore_barrier()
    _sc0.__exit__(None, None, None)

    _sc1 = jax.named_scope("p1_deg"); _sc1.__enter__()
    def deg_wave(w, carry):
        descs = []
        for j in range(DEG_F):
            k = w * DEG_F + j
            descs.append(pltpu.async_copy(
                ones_v, dego_sh.at[sid_v.at[k]], dsem, add=True))
            descs.append(pltpu.async_copy(
                ones_v, degi_sh.at[did_v.at[k]], dsem, add=True))
        for d in descs:
            d.wait()
        return carry
    lax.fori_loop(0, NCHUNK // DEG_F, deg_wave, 0)
    plsc.subcore_barrier()
    _sc1.__exit__(None, None, None)

    _sc2 = jax.named_scope("p2_norm"); _sc2.__enter__()
    def newton(src_sh, dst_ref):
        pltpu.sync_copy(src_sh.at[pl.ds(r0, ROWS_T)], db_v)
        def body(i, carry):
            d = jnp.maximum(db_v[pl.ds(i * 16, 16)], 1.0)
            s = 0.5 * (d + 1.0)
            for _ in range(15):
                s = 0.5 * (s + d / s)
            dst_ref[pl.ds(i * 16, 16)] = 1.0 / s
            return carry
        lax.fori_loop(0, ROWS_T // 16, body, 0)
    newton(dego_sh, ns_v)
    newton(degi_sh, nd_v)

    def scale_rows(norm_ref, off):
        def body(g, carry):
            nv16 = norm_ref[pl.ds(off + g * 16, 16)]
            for r16 in range(16):
                nv = jnp.full((16,), nv16[r16], jnp.float32)
                r = g * 16 + r16
                for cc in range(D_HALF // 16):
                    v = xbuf[r, pl.ds(cc * 16, 16)]
                    xbuf[r, pl.ds(cc * 16, 16)] = v * nv
            return carry
        lax.fori_loop(0, XROWS // 16, body, 0)

    for half in range(ROWS_T // XROWS):
        hr0 = r0 + half * XROWS
        off = half * XROWS
        if half < 2:
            pltpu.sync_copy(
                xs.at[pl.ds(hr0, XROWS), pl.ds(c * D_HALF, D_HALF)], xbuf)
            scale_rows(ns_v, off)
            pltpu.sync_copy(xbuf, h_hbm.at[pl.ds(c * N_PAD + hr0, XROWS)])
        else:
            @pl.when(t < NS - 1)
            def _():
                pltpu.sync_copy(
                    xs.at[pl.ds(hr0, XROWS), pl.ds(c * D_HALF, D_HALF)],
                    xbuf)
                scale_rows(ns_v, off)
                pltpu.sync_copy(xbuf,
                                h_hbm.at[pl.ds(c * N_PAD + hr0, XROWS)])
            if half == 2:
                @pl.when(t == NS - 1)
                def _():
                    p0 = N_NODES - PART
                    pltpu.sync_copy(
                        xs.at[pl.ds(p0, PART), pl.ds(c * D_HALF, D_HALF)],
                        xbuf.at[pl.ds(0, PART)])
                    scale_rows(ns_v, off)
                    pltpu.sync_copy(
                        xbuf.at[pl.ds(0, PART)],
                        h_hbm.at[pl.ds(c * N_PAD + p0, PART)])

    def adj_sid(k, carry):
        for j in range(CHUNK // 16):
            v = sid_v[k, pl.ds(j * 16, 16)]
            sid_v[k, pl.ds(j * 16, 16)] = v + c * N_PAD
        return carry
    lax.fori_loop(0, NCHUNK, adj_sid, 0)
    plsc.subcore_barrier()
    _sc2.__exit__(None, None, None)

    _sc3 = jax.named_scope("p3_edges"); _sc3.__enter__()
    def issue_g(k, b):
        pltpu.async_copy(h_hbm.at[sid_v.at[k]], rows_v.at[b], gsem)

    def wait_g(k, b):
        pltpu.make_async_copy(h_hbm.at[sid_v.at[k]], rows_v.at[b], gsem).wait()

    def issue_s(k, b):
        pltpu.async_copy(rows_v.at[b], agg_sh.at[did_v.at[k]], ssem, add=True)

    def wait_s(k, b):
        pltpu.make_async_copy(rows_v.at[b], agg_sh.at[did_v.at[k]],
                              ssem).wait()

    issue_g(0, 0)
    issue_g(1, 1)

    def edge_quad(i, carry):
        for j in range(4):
            k = 4 * i + j
            wait_g(k, j)
            @pl.when(k >= 2)
            def _():
                wait_s(k - 2, (j + 2) % 4)
            @pl.when(k + 2 < NCHUNK)
            def _():
                issue_g(k + 2, (j + 2) % 4)
            issue_s(k, j)
        return carry
    lax.fori_loop(0, NQUAD, edge_quad, 0)
    wait_g(NCHUNK - 2, 0)
    wait_s(NCHUNK - 4, 2)
    issue_s(NCHUNK - 2, 0)
    wait_g(NCHUNK - 1, 1)
    wait_s(NCHUNK - 3, 3)
    issue_s(NCHUNK - 1, 1)
    wait_s(NCHUNK - 2, 0)
    wait_s(NCHUNK - 1, 1)
    plsc.subcore_barrier()
    _sc3.__exit__(None, None, None)

    _sc4 = jax.named_scope("p4_out"); _sc4.__enter__()
    for half in range(ROWS_T // XROWS):
        hr0 = r0 + half * XROWS
        off = half * XROWS
        if half < 2:
            pltpu.sync_copy(agg_sh.at[pl.ds(hr0, XROWS)], xbuf)
            scale_rows(nd_v, off)
            pltpu.sync_copy(xbuf, out.at[c, pl.ds(hr0, XROWS)])
        else:
            @pl.when(t < NS - 1)
            def _():
                pltpu.sync_copy(agg_sh.at[pl.ds(hr0, XROWS)], xbuf)
                scale_rows(nd_v, off)
                pltpu.sync_copy(xbuf, out.at[c, pl.ds(hr0, XROWS)])
            if half == 2:
                @pl.when(t == NS - 1)
                def _():
                    p0 = N_NODES - PART
                    pltpu.sync_copy(agg_sh.at[pl.ds(p0, PART)],
                                    xbuf.at[pl.ds(0, PART)])
                    scale_rows(nd_v, off)
                    pltpu.sync_copy(xbuf.at[pl.ds(0, PART)],
                                    out.at[c, pl.ds(p0, PART)])
    _sc4.__exit__(None, None, None)


_sc_call = pl.kernel(
    _sc_body,
    out_type=(jax.ShapeDtypeStruct((NC, N_PAD, D_HALF), jnp.float32),
              jax.ShapeDtypeStruct((NC * N_PAD, D_HALF), jnp.float32)),
    mesh=plsc.VectorSubcoreMesh(core_axis_name="c", subcore_axis_name="s",
                                num_cores=NC, num_subcores=NS),
    compiler_params=pltpu.CompilerParams(use_tc_tiling_on_sc=False),
    scratch_types=[
        pltpu.VMEM((NCHUNK, CHUNK), jnp.int32),
        pltpu.VMEM((NCHUNK, CHUNK), jnp.int32),
        pltpu.VMEM((4, CHUNK, D_HALF), jnp.float32),
        pltpu.VMEM((XROWS, D_HALF), jnp.float32),
        pltpu.VMEM((ROWS_T,), jnp.float32),
        pltpu.VMEM((ROWS_T,), jnp.float32),
        pltpu.VMEM((ROWS_T,), jnp.float32),
        pltpu.VMEM((CHUNK,), jnp.float32),
        pltpu.VMEM_SHARED((N_PAD, D_HALF), jnp.float32),
        pltpu.VMEM_SHARED((N_PAD,), jnp.float32),
        pltpu.VMEM_SHARED((N_PAD,), jnp.float32),
        pltpu.SemaphoreType.DMA,
        pltpu.SemaphoreType.DMA,
        pltpu.SemaphoreType.DMA,
    ],
)

NROWP = N_PAD // 2
D2 = 2 * D_HALF
DO2 = 2 * D_OUT


def _tc_body(agg_ref, wb_ref, b1_ref, wc_ref, bc_ref, out_ref):
    h = jnp.dot(agg_ref[0], wb_ref[0], preferred_element_type=jnp.float32)
    h = h + jnp.dot(agg_ref[1], wb_ref[1], preferred_element_type=jnp.float32)
    h = jnp.maximum(h + b1_ref[...], 0.0)
    mask = lax.broadcasted_iota(jnp.int32, (NROWP, 1), 0) < N_NODES // 2
    h = jnp.where(mask, h, 0.0)
    s = jnp.sum(h, axis=0, keepdims=True)
    g = (s[:, :D_OUT] + s[:, D_OUT:]) * (1.0 / N_NODES)
    out_ref[...] = (jnp.dot(g, wc_ref[...], preferred_element_type=jnp.float32)
                    + bc_ref[...])


def kernel(x, edge_index, W1, b1, Wc, bc):
    er = edge_index.reshape(2, NS, NCHUNK, CHUNK)
    agg, _ = _sc_call(x, er)
    aggp = agg.reshape(NC, NROWP, D2)
    w1c = W1.reshape(NC, D_HALF, D_OUT)
    z = jnp.zeros((NC, D_HALF, D_OUT), jnp.float32)
    wb = jnp.concatenate([jnp.concatenate([w1c, z], axis=2),
                          jnp.concatenate([z, w1c], axis=2)], axis=1)
    b1b = jnp.concatenate([b1, b1]).reshape(1, DO2)
    return pl.pallas_call(
        _tc_body,
        out_shape=jax.ShapeDtypeStruct((1, N_CLASSES), jnp.float32),
    )(aggp, wb, b1b, Wc, bc.reshape(1, N_CLASSES))


# speedup vs baseline: 1.1346x; 1.1346x over previous
import jax
import jax.numpy as jnp
from jax import lax
from jax.experimental import pallas as pl
from jax.experimental.pallas import tpu as pltpu
from jax.experimental.pallas import tpu_sc as plsc

N_NODES = 10000
N_PAD = 10240
N_EDGES = 320000
D_IN = 128
D_HALF = 64
D_OUT = 256
N_CLASSES = 2
NC = 2
NS = 16
ROWS_T = N_PAD // NS
CHUNK = 128
NCH_ALL = N_EDGES // CHUNK
NCHUNK = NCH_ALL // NS
N_EXTRA = NCH_ALL - NS * NCHUNK
DEG_F = 12
XROWS = 160
PART = N_NODES - (NS - 1) * ROWS_T - 2 * XROWS


def _sc_body(xs, er, out, h_hbm,
             sid_v, did_v, rows_v, xbuf, ns_v, nd_v, db_v, ones_v,
             agg_sh, dego_sh, degi_sh, gsem, ssem, dsem):
    c = lax.axis_index("c")
    t = lax.axis_index("s")
    r0 = t * ROWS_T

    _sc0 = jax.named_scope("p0_setup"); _sc0.__enter__()
    pltpu.sync_copy(er.at[0, pl.ds(t * NCHUNK, NCHUNK)],
                    sid_v.at[pl.ds(0, NCHUNK)])
    pltpu.sync_copy(er.at[1, pl.ds(t * NCHUNK, NCHUNK)],
                    did_v.at[pl.ds(0, NCHUNK)])
    @pl.when(t < N_EXTRA)
    def _():
        pltpu.sync_copy(er.at[0, pl.ds(NS * NCHUNK + t, 1)],
                        sid_v.at[pl.ds(NCHUNK, 1)])
        pltpu.sync_copy(er.at[1, pl.ds(NS * NCHUNK + t, 1)],
                        did_v.at[pl.ds(NCHUNK, 1)])

    def zero_xbuf(i, carry):
        for j in range(D_HALF // 16):
            xbuf[i, pl.ds(j * 16, 16)] = jnp.zeros((16,), jnp.float32)
        return carry
    lax.fori_loop(0, XROWS, zero_xbuf, 0)

    def zero_db(i, carry):
        db_v[pl.ds(i * 16, 16)] = jnp.zeros((16,), jnp.float32)
        return carry
    lax.fori_loop(0, ROWS_T // 16, zero_db, 0)

    def fill_ones(i, carry):
        ones_v[pl.ds(i * 16, 16)] = jnp.full((16,), 1.0, jnp.float32)
        return carry
    lax.fori_loop(0, CHUNK // 16, fill_ones, 0)

    for q in range(ROWS_T // XROWS):
        pltpu.sync_copy(xbuf, agg_sh.at[pl.ds(r0 + q * XROWS, XROWS)])
    pltpu.sync_copy(db_v, dego_sh.at[pl.ds(r0, ROWS_T)])
    pltpu.sync_copy(db_v, degi_sh.at[pl.ds(r0, ROWS_T)])
    plsc.subcore_barrier()
    _sc0.__exit__(None, None, None)

    _sc1 = jax.named_scope("p1_deg"); _sc1.__enter__()
    def deg_wave(w, carry):
        descs = []
        for j in range(DEG_F):
            k = w * DEG_F + j
            descs.append(pltpu.async_copy(
                ones_v, dego_sh.at[sid_v.at[k]], dsem, add=True))
            descs.append(pltpu.async_copy(
                ones_v, degi_sh.at[did_v.at[k]], dsem, add=True))
        for d in descs:
            d.wait()
        return carry
    lax.fori_loop(0, NCHUNK // DEG_F, deg_wave, 0)
    @pl.when(t < N_EXTRA)
    def _():
        d1 = pltpu.async_copy(ones_v, dego_sh.at[sid_v.at[NCHUNK]],
                              dsem, add=True)
        d2 = pltpu.async_copy(ones_v, degi_sh.at[did_v.at[NCHUNK]],
                              dsem, add=True)
        d1.wait()
        d2.wait()
    plsc.subcore_barrier()
    _sc1.__exit__(None, None, None)

    _sc2 = jax.named_scope("p2_norm"); _sc2.__enter__()
    def newton(src_sh, dst_ref):
        pltpu.sync_copy(src_sh.at[pl.ds(r0, ROWS_T)], db_v)
        def body(i, carry):
            d = jnp.maximum(db_v[pl.ds(i * 16, 16)], 1.0)
            s = 0.5 * (d + 1.0)
            for _ in range(15):
                s = 0.5 * (s + d / s)
            dst_ref[pl.ds(i * 16, 16)] = 1.0 / s
            return carry
        lax.fori_loop(0, ROWS_T // 16, body, 0)
    newton(dego_sh, ns_v)
    newton(degi_sh, nd_v)

    def scale_rows(norm_ref, off):
        def body(g, carry):
            nv16 = norm_ref[pl.ds(off + g * 16, 16)]
            for r16 in range(16):
                nv = jnp.full((16,), nv16[r16], jnp.float32)
                r = g * 16 + r16
                for cc in range(D_HALF // 16):
                    v = xbuf[r, pl.ds(cc * 16, 16)]
                    xbuf[r, pl.ds(cc * 16, 16)] = v * nv
            return carry
        lax.fori_loop(0, XROWS // 16, body, 0)

    for half in range(ROWS_T // XROWS):
        hr0 = r0 + half * XROWS
        off = half * XROWS
        if half < 2:
            pltpu.sync_copy(
                xs.at[pl.ds(hr0, XROWS), pl.ds(c * D_HALF, D_HALF)], xbuf)
            scale_rows(ns_v, off)
            pltpu.sync_copy(xbuf, h_hbm.at[pl.ds(c * N_PAD + hr0, XROWS)])
        else:
            @pl.when(t < NS - 1)
            def _():
                pltpu.sync_copy(
                    xs.at[pl.ds(hr0, XROWS), pl.ds(c * D_HALF, D_HALF)],
                    xbuf)
                scale_rows(ns_v, off)
                pltpu.sync_copy(xbuf,
                                h_hbm.at[pl.ds(c * N_PAD + hr0, XROWS)])
            if half == 2:
                @pl.when(t == NS - 1)
                def _():
                    p0 = N_NODES - PART
                    pltpu.sync_copy(
                        xs.at[pl.ds(p0, PART), pl.ds(c * D_HALF, D_HALF)],
                        xbuf.at[pl.ds(0, PART)])
                    scale_rows(ns_v, off)
                    pltpu.sync_copy(
                        xbuf.at[pl.ds(0, PART)],
                        h_hbm.at[pl.ds(c * N_PAD + p0, PART)])

    def adj_sid(k, carry):
        for j in range(CHUNK // 16):
            v = sid_v[k, pl.ds(j * 16, 16)]
            sid_v[k, pl.ds(j * 16, 16)] = v + c * N_PAD
        return carry
    lax.fori_loop(0, NCHUNK + 1, adj_sid, 0)
    plsc.subcore_barrier()
    _sc2.__exit__(None, None, None)

    _sc3 = jax.named_scope("p3_edges"); _sc3.__enter__()
    def issue_g(k, b):
        pltpu.async_copy(h_hbm.at[sid_v.at[k]], rows_v.at[b], gsem)

    def wait_g(k, b):
        pltpu.make_async_copy(h_hbm.at[sid_v.at[k]], rows_v.at[b], gsem).wait()

    def issue_s(k, b):
        pltpu.async_copy(rows_v.at[b], agg_sh.at[did_v.at[k]], ssem, add=True)

    def wait_s(k, b):
        pltpu.make_async_copy(rows_v.at[b], agg_sh.at[did_v.at[k]],
                              ssem).wait()

    issue_g(0, 0)
    issue_g(1, 1)

    def edge_quad(i, carry):
        for j in range(4):
            k = 4 * i + j
            wait_g(k, j)
            @pl.when(k >= 2)
            def _():
                wait_s(k - 2, (j + 2) % 4)
            @pl.when(k + 2 < NCHUNK)
            def _():
                issue_g(k + 2, (j + 2) % 4)
            issue_s(k, j)
        return carry
    lax.fori_loop(0, NCHUNK // 4, edge_quad, 0)
    wait_s(NCHUNK - 2, 2)
    wait_s(NCHUNK - 1, 3)
    @pl.when(t < N_EXTRA)
    def _():
        pltpu.async_copy(h_hbm.at[sid_v.at[NCHUNK]], rows_v.at[0],
                         gsem).wait()
        pltpu.sync_copy(rows_v.at[0], agg_sh.at[did_v.at[NCHUNK]], add=True)
    plsc.subcore_barrier()
    _sc3.__exit__(None, None, None)

    _sc4 = jax.named_scope("p4_out"); _sc4.__enter__()
    for half in range(ROWS_T // XROWS):
        hr0 = r0 + half * XROWS
        off = half * XROWS
        if half < 2:
            pltpu.sync_copy(agg_sh.at[pl.ds(hr0, XROWS)], xbuf)
            scale_rows(nd_v, off)
            pltpu.sync_copy(xbuf, out.at[c, pl.ds(hr0, XROWS)])
        else:
            @pl.when(t < NS - 1)
            def _():
                pltpu.sync_copy(agg_sh.at[pl.ds(hr0, XROWS)], xbuf)
                scale_rows(nd_v, off)
                pltpu.sync_copy(xbuf, out.at[c, pl.ds(hr0, XROWS)])
            if half == 2:
                @pl.when(t == NS - 1)
                def _():
                    p0 = N_NODES - PART
                    pltpu.sync_copy(agg_sh.at[pl.ds(p0, PART)],
                                    xbuf.at[pl.ds(0, PART)])
                    scale_rows(nd_v, off)
                    pltpu.sync_copy(xbuf.at[pl.ds(0, PART)],
                                    out.at[c, pl.ds(p0, PART)])
    _sc4.__exit__(None, None, None)


_sc_call = pl.kernel(
    _sc_body,
    out_type=(jax.ShapeDtypeStruct((NC, N_PAD, D_HALF), jnp.float32),
              jax.ShapeDtypeStruct((NC * N_PAD, D_HALF), jnp.float32)),
    mesh=plsc.VectorSubcoreMesh(core_axis_name="c", subcore_axis_name="s",
                                num_cores=NC, num_subcores=NS),
    compiler_params=pltpu.CompilerParams(use_tc_tiling_on_sc=False),
    scratch_types=[
        pltpu.VMEM((NCHUNK + 1, CHUNK), jnp.int32),
        pltpu.VMEM((NCHUNK + 1, CHUNK), jnp.int32),
        pltpu.VMEM((4, CHUNK, D_HALF), jnp.float32),
        pltpu.VMEM((XROWS, D_HALF), jnp.float32),
        pltpu.VMEM((ROWS_T,), jnp.float32),
        pltpu.VMEM((ROWS_T,), jnp.float32),
        pltpu.VMEM((ROWS_T,), jnp.float32),
        pltpu.VMEM((CHUNK,), jnp.float32),
        pltpu.VMEM_SHARED((N_PAD, D_HALF), jnp.float32),
        pltpu.VMEM_SHARED((N_PAD,), jnp.float32),
        pltpu.VMEM_SHARED((N_PAD,), jnp.float32),
        pltpu.SemaphoreType.DMA,
        pltpu.SemaphoreType.DMA,
        pltpu.SemaphoreType.DMA,
    ],
)

NROWP = N_PAD // 2
D2 = 2 * D_HALF
DO2 = 2 * D_OUT


def _tc_body(agg_ref, wb_ref, b1_ref, wc_ref, bc_ref, out_ref):
    h = jnp.dot(agg_ref[0], wb_ref[0], preferred_element_type=jnp.float32)
    h = h + jnp.dot(agg_ref[1], wb_ref[1], preferred_element_type=jnp.float32)
    h = jnp.maximum(h + b1_ref[...], 0.0)
    mask = lax.broadcasted_iota(jnp.int32, (NROWP, 1), 0) < N_NODES // 2
    h = jnp.where(mask, h, 0.0)
    s = jnp.sum(h, axis=0, keepdims=True)
    g = (s[:, :D_OUT] + s[:, D_OUT:]) * (1.0 / N_NODES)
    out_ref[...] = (jnp.dot(g, wc_ref[...], preferred_element_type=jnp.float32)
                    + bc_ref[...])


def kernel(x, edge_index, W1, b1, Wc, bc):
    er = edge_index.reshape(2, NCH_ALL, CHUNK)
    agg, _ = _sc_call(x, er)
    aggp = agg.reshape(NC, NROWP, D2)
    w1c = W1.reshape(NC, D_HALF, D_OUT)
    z = jnp.zeros((NC, D_HALF, D_OUT), jnp.float32)
    wb = jnp.concatenate([jnp.concatenate([w1c, z], axis=2),
                          jnp.concatenate([z, w1c], axis=2)], axis=1)
    b1b = jnp.concatenate([b1, b1]).reshape(1, DO2)
    return pl.pallas_call(
        _tc_body,
        out_shape=jax.ShapeDtypeStruct((1, N_CLASSES), jnp.float32),
    )(aggp, wb, b1b, Wc, bc.reshape(1, N_CLASSES))
